# baseline (speedup 1.0000x reference)
import functools

import jax
import jax.numpy as jnp
from jax import lax
from jax.experimental import pallas as pl
from jax.experimental.pallas import tpu as pltpu
from jax.experimental.pallas import tpu_sc as plsc

B, P, D = 32, 576, 768
NW = 32
R1 = 16
R2 = 8
NCOL = D // 16
NB = 8
LA = 4
NG = B // NB

_mesh = plsc.VectorSubcoreMesh(core_axis_name="c", subcore_axis_name="s")


@functools.partial(
    pl.kernel,
    mesh=_mesh,
    out_type=jax.ShapeDtypeStruct((B, P, D), jnp.float32),
    scratch_types=(
        [pltpu.VMEM((R1, D), jnp.float32)]
        + [pltpu.VMEM((R2, D), jnp.float32)]
        + [pltpu.VMEM((R1, D), jnp.float32)] * NB
        + [pltpu.VMEM((R2, D), jnp.float32)] * 2
        + [pltpu.SemaphoreType.DMA] * (2 * NB + 4 + 1)
    ),
)
def _sc_add(x_hbm, emb_hbm, out_hbm, emb1, emb2, *rest):
    bufs1 = rest[:NB]
    bufs2 = rest[NB:NB + 2]
    sems = rest[NB + 2:]
    l1 = sems[:NB]
    s1 = sems[NB:2 * NB]
    l2 = sems[2 * NB:2 * NB + 2]
    s2 = sems[2 * NB + 2:2 * NB + 4]
    le = sems[2 * NB + 4]

    wid = lax.axis_index("s") * 2 + lax.axis_index("c")
    rb1 = wid * R1
    rb2 = NW * R1 + (wid // 4) * R2
    tb0 = (wid % 4) * 8

    def load1(b, j):
        pltpu.async_copy(x_hbm.at[b, pl.ds(rb1, R1), :], bufs1[j], l1[j])

    def load2(i, u):
        pltpu.async_copy(x_hbm.at[tb0 + i, pl.ds(rb2, R2), :], bufs2[u], l2[u])

    def add_emb(buf, emb_v, nrows):
        def body(r, _):
            for c in range(NCOL):
                s = pl.ds(c * 16, 16)
                plsc.addupdate(buf.at[r, s], emb_v[r, s])
            return ()
        lax.fori_loop(0, nrows, body, ())

    pltpu.async_copy(emb_hbm.at[pl.ds(rb1, R1), :], emb1, le)
    for b in range(LA):
        load1(b, b)
    load2(0, 0)
    load2(1, 1)
    pltpu.make_async_copy(emb_hbm.at[pl.ds(rb1, R1), :], emb1, le).wait()
    pltpu.async_copy(emb_hbm.at[pl.ds(rb2, R2), :], emb2, le)
    pltpu.make_async_copy(emb_hbm.at[pl.ds(rb2, R2), :], emb2, le).wait()

    def group(g, _):
        for j in range(NB):
            b = g * NB + j
            jn = (j + LA) % NB

            @pl.when(b >= LA)
            def _():
                pltpu.make_async_copy(
                    bufs1[jn], out_hbm.at[b - LA, pl.ds(rb1, R1), :],
                    s1[jn]).wait()

            @pl.when(b + LA < B)
            def _():
                load1(b + LA, jn)

            pltpu.make_async_copy(
                x_hbm.at[b, pl.ds(rb1, R1), :], bufs1[j], l1[j]).wait()
            add_emb(bufs1[j], emb1, R1)
            pltpu.async_copy(
                bufs1[j], out_hbm.at[b, pl.ds(rb1, R1), :], s1[j])

            if j == 1:
                pltpu.make_async_copy(
                    x_hbm.at[tb0 + 2 * g, pl.ds(rb2, R2), :], bufs2[0],
                    l2[0]).wait()
                add_emb(bufs2[0], emb2, R2)
                pltpu.async_copy(
                    bufs2[0], out_hbm.at[tb0 + 2 * g, pl.ds(rb2, R2), :],
                    s2[0])
            if j == 3:
                pltpu.make_async_copy(
                    bufs2[0], out_hbm.at[tb0 + 2 * g, pl.ds(rb2, R2), :],
                    s2[0]).wait()

                @pl.when(2 * g + 2 < 2 * NG)
                def _():
                    load2(2 * g + 2, 0)
            if j == 5:
                pltpu.make_async_copy(
                    x_hbm.at[tb0 + 2 * g + 1, pl.ds(rb2, R2), :], bufs2[1],
                    l2[1]).wait()
                add_emb(bufs2[1], emb2, R2)
                pltpu.async_copy(
                    bufs2[1], out_hbm.at[tb0 + 2 * g + 1, pl.ds(rb2, R2), :],
                    s2[1])
            if j == 7:
                pltpu.make_async_copy(
                    bufs2[1], out_hbm.at[tb0 + 2 * g + 1, pl.ds(rb2, R2), :],
                    s2[1]).wait()

                @pl.when(2 * g + 3 < 2 * NG)
                def _():
                    load2(2 * g + 3, 1)
        return ()

    lax.fori_loop(0, NG, group, ())

    for b in range(B - LA, B):
        pltpu.make_async_copy(
            bufs1[b % NB], out_hbm.at[b, pl.ds(rb1, R1), :], s1[b % NB]).wait()


def kernel(x, embedding):
    return _sc_add(x, embedding)
